# baseline (speedup 1.0000x reference)
import jax
import jax.numpy as jnp
from jax import lax
from jax.experimental import pallas as pl
from jax.experimental.pallas import tpu as pltpu
from jax.experimental.pallas import tpu_sc as plsc

NUM_RANKS = 1000
NUM_CTX = 16
NUM_RANK_TOK = 4
MAX_TOK = 77
DIM = 768

_NC = 2
_NS = 16
_NW = _NC * _NS
_BASE = NUM_RANKS // _NW
_EXTRA = NUM_RANKS - _BASE * _NW
_HEAD = 24
_TAILN = _BASE + 1


def _vrow_copy(dst, drow, src, srow):
    for c in range(0, DIM, 16):
        dst[drow, pl.ds(c, 16)] = src[srow, pl.ds(c, 16)]


def _sc_body(ctx_hbm, rank_hbm, sent_hbm, out_hbm,
             head, ctxs, ranks, row8, sem):
    wid = lax.axis_index("s") * _NC + lax.axis_index("c")
    start = _BASE * wid + jnp.minimum(wid, _EXTRA)
    cnt = jnp.where(wid < _EXTRA, _BASE + 1, _BASE)

    tstart = jnp.minimum(start, NUM_RANKS - _TAILN)
    tail_cp = pltpu.make_async_copy(
        sent_hbm.at[pl.ds(tstart, _TAILN), pl.ds(_HEAD, MAX_TOK - _HEAD)],
        out_hbm.at[pl.ds(tstart, _TAILN), pl.ds(_HEAD, MAX_TOK - _HEAD)],
        sem)
    tail_cp.start()

    pltpu.sync_copy(ctx_hbm, ctxs)
    for j in range(NUM_CTX):
        _vrow_copy(head, 1 + j, ctxs, j)

    def body(i, carry):
        r = start + i
        pltpu.sync_copy(sent_hbm.at[r, pl.ds(0, 1)], head.at[pl.ds(0, 1)])
        pltpu.sync_copy(rank_hbm.at[r], ranks)
        for j in range(NUM_RANK_TOK):
            _vrow_copy(head, 1 + NUM_CTX + j, ranks, j)
        pltpu.sync_copy(sent_hbm.at[r, pl.ds(16, 8)], row8)
        for j in range(3):
            _vrow_copy(head, 21 + j, row8, 5 + j)
        pltpu.sync_copy(head, out_hbm.at[r, pl.ds(0, _HEAD)])
        return carry

    lax.fori_loop(0, cnt, body, 0)
    tail_cp.wait()


def kernel(context_embeds, rank_embeds, sentence_embeds):
    run = pl.kernel(
        _sc_body,
        out_type=jax.ShapeDtypeStruct((NUM_RANKS, MAX_TOK, DIM), jnp.float32),
        mesh=plsc.VectorSubcoreMesh(core_axis_name="c", subcore_axis_name="s"),
        scratch_types=[
            pltpu.VMEM((_HEAD, DIM), jnp.float32),
            pltpu.VMEM((NUM_CTX, DIM), jnp.float32),
            pltpu.VMEM((NUM_RANK_TOK, DIM), jnp.float32),
            pltpu.VMEM((8, DIM), jnp.float32),
            pltpu.SemaphoreType.DMA,
        ],
    )
    return run(context_embeds, rank_embeds, sentence_embeds)
